# Initial kernel scaffold; baseline (speedup 1.0000x reference)
import jax
import jax.numpy as jnp
from jax.experimental import pallas as pl


def kernel(x, weight, lora_A, lora_B):
    raise NotImplementedError("write your pallas kernel here")


# speedup vs baseline: 4.1670x; 4.1670x over previous
import functools

import jax
import jax.numpy as jnp
from jax import lax
from jax.experimental import pallas as pl
from jax.experimental.pallas import tpu as pltpu
from jax.experimental.pallas import tpu_sc as plsc

M = 8
R = 8
V = 100000
D = 64
B = 1024
T = 50
LORA_ALPHA = 16.0
SCALING = LORA_ALPHA / R

N = B * T
NC, NS = 2, 16
NW = NC * NS
B_PER_W = N // NW
CH = 128


def _sc_gather(idx_flat, weight, a2):
    mesh = plsc.VectorSubcoreMesh(core_axis_name="c", subcore_axis_name="s")
    n_chunks = (B_PER_W + CH - 1) // CH

    @functools.partial(
        pl.kernel,
        mesh=mesh,
        compiler_params=pltpu.CompilerParams(use_tc_tiling_on_sc=False),
        out_type=[
            jax.ShapeDtypeStruct((N, D), jnp.float32),
            jax.ShapeDtypeStruct((N, M * R), jnp.float32),
        ],
        scratch_types=[
            pltpu.VMEM((B_PER_W,), jnp.int32),
            pltpu.VMEM((B_PER_W, D), jnp.float32),
            pltpu.SemaphoreType.DMA,
        ],
    )
    def gather_kernel(idx_hbm, w_hbm, a2_hbm, gw_hbm, ga_hbm, idx_v, rows_v, sem):
        wid = lax.axis_index("s") * NC + lax.axis_index("c")
        base = wid * B_PER_W
        pltpu.sync_copy(idx_hbm.at[pl.ds(base, B_PER_W)], idx_v)
        for tbl, out_hbm in ((w_hbm, gw_hbm), (a2_hbm, ga_hbm)):
            copies = []
            for c in range(n_chunks):
                lo = c * CH
                sz = min(CH, B_PER_W - lo)
                copies.append(pltpu.async_copy(
                    tbl.at[idx_v.at[pl.ds(lo, sz)]],
                    rows_v.at[pl.ds(lo, sz)],
                    sem,
                ))
            for cp in copies:
                cp.wait()
            pltpu.sync_copy(rows_v, out_hbm.at[pl.ds(base, B_PER_W)])

    return gather_kernel(idx_flat, weight, a2)


def _tc_combine(gw4, ga4, q4):
    n4 = gw4.shape[0]
    tn4 = 512
    grid = (n4 // tn4,)

    def body(gw_ref, ga_ref, q_ref, out_ref):
        gw = gw_ref[...]
        ga = ga_ref[...]
        for m in range(M):
            out_ref[m] = gw + jnp.dot(ga, q_ref[m],
                                      preferred_element_type=jnp.float32)

    return pl.pallas_call(
        body,
        grid=grid,
        in_specs=[
            pl.BlockSpec((tn4, 4 * D), lambda i: (i, 0)),
            pl.BlockSpec((tn4, 4 * D), lambda i: (i, 0)),
            pl.BlockSpec((M, 4 * D, 4 * D), lambda i: (0, 0, 0)),
        ],
        out_specs=pl.BlockSpec((M, tn4, 4 * D), lambda i: (0, i, 0)),
        out_shape=jax.ShapeDtypeStruct((M, n4, 4 * D), jnp.float32),
    )(gw4, ga4, q4)


def kernel(x, weight, lora_A, lora_B):
    idx_flat = x.reshape(N)
    a2 = lora_A.reshape(M * R, V).T

    gw, ga = _sc_gather(idx_flat, weight, a2)

    p = SCALING * jnp.transpose(lora_B, (0, 2, 1))
    p_tiled = jnp.tile(p, (1, M, 1))
    sel = (jnp.arange(M * R)[None, :, None] // R
           == jnp.arange(M)[:, None, None])
    p_big = jnp.where(sel, p_tiled, 0.0)
    q4 = jax.vmap(lambda pm: jnp.kron(jnp.eye(4, dtype=pm.dtype), pm))(p_big)

    n4 = N // 4
    out4 = _tc_combine(gw.reshape(n4, 4 * D), ga.reshape(n4, 4 * D), q4)
    return out4.reshape(M, B, T, D)


# speedup vs baseline: 4.2420x; 1.0180x over previous
import functools

import jax
import jax.numpy as jnp
from jax import lax
from jax.experimental import pallas as pl
from jax.experimental.pallas import tpu as pltpu
from jax.experimental.pallas import tpu_sc as plsc

M = 8
R = 8
V = 100000
D = 64
B = 1024
T = 50
LORA_ALPHA = 16.0
SCALING = LORA_ALPHA / R

N = B * T
NC, NS = 2, 16
NW = NC * NS
B_PER_W = N // NW
HALF = B_PER_W // 2
CH = 128


def _tc_prep(weight, lora_a_flat):
    vt = 1024
    grid = (pl.cdiv(V, vt),)

    def body(w_ref, a_ref, out_ref):
        out_ref[:, :D] = w_ref[...]
        out_ref[:, D:] = jnp.transpose(a_ref[...], (1, 0))

    return pl.pallas_call(
        body,
        grid=grid,
        in_specs=[
            pl.BlockSpec((vt, D), lambda i: (i, 0)),
            pl.BlockSpec((M * R, vt), lambda i: (0, i)),
        ],
        out_specs=pl.BlockSpec((vt, 2 * D), lambda i: (i, 0)),
        out_shape=jax.ShapeDtypeStruct((V, 2 * D), jnp.float32),
    )(weight, lora_a_flat)


def _sc_gather(idx_flat, wc):
    mesh = plsc.VectorSubcoreMesh(core_axis_name="c", subcore_axis_name="s")

    @functools.partial(
        pl.kernel,
        mesh=mesh,
        compiler_params=pltpu.CompilerParams(use_tc_tiling_on_sc=False),
        out_type=jax.ShapeDtypeStruct((N, 2 * D), jnp.float32),
        scratch_types=[
            pltpu.VMEM((B_PER_W,), jnp.int32),
            pltpu.VMEM((HALF, 2 * D), jnp.float32),
            pltpu.SemaphoreType.DMA,
        ],
    )
    def gather_kernel(idx_hbm, wc_hbm, g_hbm, idx_v, rows_v, sem):
        wid = lax.axis_index("s") * NC + lax.axis_index("c")
        base = wid * B_PER_W
        pltpu.sync_copy(idx_hbm.at[pl.ds(base, B_PER_W)], idx_v)
        for h in range(2):
            copies = []
            for lo in range(0, HALF, CH):
                sz = min(CH, HALF - lo)
                copies.append(pltpu.async_copy(
                    wc_hbm.at[idx_v.at[pl.ds(h * HALF + lo, sz)]],
                    rows_v.at[pl.ds(lo, sz)],
                    sem,
                ))
            for cp in copies:
                cp.wait()
            pltpu.sync_copy(rows_v, g_hbm.at[pl.ds(base + h * HALF, HALF)])

    return gather_kernel(idx_flat, wc)


def _tc_combine(g, qf):
    n2 = N // 2
    tn = 2048
    grid = (N // tn,)

    def body(g_ref, qf_ref, out_ref):
        g2 = g_ref[...].reshape(tn // 2, 4 * D)
        for m in range(M):
            out_ref[m] = jnp.dot(g2, qf_ref[m],
                                 preferred_element_type=jnp.float32)

    return pl.pallas_call(
        body,
        grid=grid,
        in_specs=[
            pl.BlockSpec((tn, 2 * D), lambda i: (i, 0)),
            pl.BlockSpec((M, 4 * D, 2 * D), lambda i: (0, 0, 0)),
        ],
        out_specs=pl.BlockSpec((M, tn // 2, 2 * D), lambda i: (0, i, 0)),
        out_shape=jax.ShapeDtypeStruct((M, n2, 2 * D), jnp.float32),
    )(g, qf)


def kernel(x, weight, lora_A, lora_B):
    idx_flat = x.reshape(N)
    wc = _tc_prep(weight, lora_A.reshape(M * R, V))
    g = _sc_gather(idx_flat, wc)

    p = SCALING * jnp.transpose(lora_B, (0, 2, 1))
    p_tiled = jnp.tile(p, (1, M, 1))
    sel = (jnp.arange(M * R)[None, :, None] // R
           == jnp.arange(M)[:, None, None])
    p_big = jnp.where(sel, p_tiled, 0.0)
    eye = jnp.broadcast_to(jnp.eye(D, dtype=jnp.float32), (M, D, D))
    u = jnp.concatenate([eye, p_big], axis=1)
    qf = jax.vmap(lambda um: jnp.kron(jnp.eye(2, dtype=um.dtype), um))(u)

    out2 = _tc_combine(g, qf)
    return out2.reshape(M, B, T, D)
